# baseline (speedup 1.0000x reference)
import functools

import jax
import jax.numpy as jnp
from jax.experimental import pallas as pl
from jax.experimental.pallas import tpu as pltpu


def _conv1_kernel(xe_ref, xo_ref, w1_ref, b1_ref, o_ref, *, H, W, C, B):
    Hh = H // 2
    for n in range(B):
        xe = xe_ref[n]
        xo = xo_ref[n]

        rows_e = [xe[0:Hh], xo[0:Hh], xe[1:Hh + 1]]
        rows_o = [xo[0:Hh], xe[1:Hh + 1], xo[1:Hh + 1]]
        win_e = [[r[:, kw:kw + W] for kw in range(3)] for r in rows_e]
        win_o = [[r[:, kw:kw + W] for kw in range(3)] for r in rows_o]

        for c in range(C):
            acc_e = win_e[0][0] * w1_ref[c, 0]
            acc_o = win_o[0][0] * w1_ref[c, 0]
            for kh in range(3):
                for kw in range(3):
                    if kh == 0 and kw == 0:
                        continue
                    t = kh * 3 + kw
                    acc_e = acc_e + win_e[kh][kw] * w1_ref[c, t]
                    acc_o = acc_o + win_o[kh][kw] * w1_ref[c, t]
            b = b1_ref[c]
            o_ref[n, c] = jnp.maximum(jnp.maximum(acc_e, acc_o) + b, 0.0)


def _conv1_call(xe, xo, w1s, b1, *, B=16):
    N, Hs, Wp = xe.shape
    H, W = (Hs - 1) * 2, Wp - 2
    C = w1s.shape[0]
    B = min(B, N)
    body = functools.partial(_conv1_kernel, H=H, W=W, C=C, B=B)
    return pl.pallas_call(
        body,
        out_shape=jax.ShapeDtypeStruct((N, C, H // 2, W), jnp.float32),
        grid=(N // B,),
        in_specs=[
            pl.BlockSpec((B, Hs, Wp), lambda n: (n, 0, 0)),
            pl.BlockSpec((B, Hs, Wp), lambda n: (n, 0, 0)),
            pl.BlockSpec(memory_space=pltpu.MemorySpace.SMEM),
            pl.BlockSpec(memory_space=pltpu.MemorySpace.SMEM),
        ],
        out_specs=pl.BlockSpec((B, C, H // 2, W), lambda n: (n, 0, 0, 0)),
        compiler_params=pltpu.CompilerParams(
            dimension_semantics=("parallel",),
        ),
    )(xe, xo, w1s, b1)


def _conv2_kernel(xp_ref, w_ref, b_ref, o_ref, p_ref, y_ref,
                  *, H, W, Cin, Cout, B):
    for n in range(B):
        for kh in range(3):
            for kw in range(3):
                t = kh * 3 + kw
                p_ref[:, t * Cin:(t + 1) * Cin] = (
                    xp_ref[n, kh:kh + H, kw:kw + W, :].reshape(H * W, Cin))

        y = jnp.dot(p_ref[...], w_ref[...], preferred_element_type=jnp.float32)
        y_ref[...] = jnp.maximum(y + b_ref[...], 0.0)

        hw2 = (H * W) // 2
        yw = jnp.maximum(y_ref[pl.ds(0, hw2, 2), :],
                         y_ref[pl.ds(1, hw2, 2), :])
        o_ref[n] = jnp.max(yw.reshape(H // 2, 2, W // 2, Cout),
                           axis=1).reshape((H // 2) * (W // 2), Cout)


def _conv2_call(xp_hwc, w2f, b2, *, B=4):
    N, Hp, Wp, Cin = xp_hwc.shape
    H, W = Hp - 2, Wp - 2
    Cout = w2f.shape[-1]
    B = min(B, N)
    body = functools.partial(_conv2_kernel, H=H, W=W, Cin=Cin, Cout=Cout, B=B)
    P = (H // 2) * (W // 2)
    return pl.pallas_call(
        body,
        out_shape=jax.ShapeDtypeStruct((N, P, Cout), jnp.float32),
        grid=(N // B,),
        in_specs=[
            pl.BlockSpec((B, Hp, Wp, Cin), lambda n: (n, 0, 0, 0)),
            pl.BlockSpec((9 * Cin, Cout), lambda n: (0, 0)),
            pl.BlockSpec((1, Cout), lambda n: (0, 0)),
        ],
        out_specs=pl.BlockSpec((B, P, Cout), lambda n: (n, 0, 0)),
        scratch_shapes=[
            pltpu.VMEM((H * W, 9 * Cin), jnp.float32),
            pltpu.VMEM((H * W, Cout), jnp.float32),
        ],
        compiler_params=pltpu.CompilerParams(
            dimension_semantics=("parallel",),
            vmem_limit_bytes=64 * 1024 * 1024,
        ),
    )(xp_hwc, w2f, b2)


def _fc_kernel(x_ref, w1_ref, b1_ref, w2_ref, b2_ref, o_ref, acc_ref):
    k = pl.program_id(0)

    @pl.when(k == 0)
    def _():
        acc_ref[...] = jnp.zeros_like(acc_ref)

    acc_ref[...] += jnp.dot(x_ref[...], w1_ref[...],
                            preferred_element_type=jnp.float32)

    @pl.when(k == pl.num_programs(0) - 1)
    def _():
        h = jnp.maximum(acc_ref[...] + b1_ref[...], 0.0)
        o_ref[...] = (jnp.dot(h, w2_ref[...],
                              preferred_element_type=jnp.float32)
                      + b2_ref[...])


def _fc_call(x, fw1, fb1, fw2, fb2, *, tk=4096):
    N, K = x.shape
    tk = min(tk, K)
    Dh = fw1.shape[1]
    Do = fw2.shape[1]
    return pl.pallas_call(
        _fc_kernel,
        out_shape=jax.ShapeDtypeStruct((N, Do), jnp.float32),
        grid=(K // tk,),
        in_specs=[
            pl.BlockSpec((N, tk), lambda k: (0, k)),
            pl.BlockSpec((tk, Dh), lambda k: (k, 0)),
            pl.BlockSpec((1, Dh), lambda k: (0, 0)),
            pl.BlockSpec((Dh, Do), lambda k: (0, 0)),
            pl.BlockSpec((1, Do), lambda k: (0, 0)),
        ],
        out_specs=pl.BlockSpec((N, Do), lambda k: (0, 0)),
        scratch_shapes=[pltpu.VMEM((N, Dh), jnp.float32)],
        compiler_params=pltpu.CompilerParams(
            dimension_semantics=("arbitrary",),
        ),
    )(x, fw1, fb1, fw2, fb2)


@jax.jit
def _forward(x_nchw, w1, b1, w2, b2, fw1, fb1, fw2, fb2):
    N = x_nchw.shape[0]
    xp = jnp.pad(x_nchw[:, 0], ((0, 0), (1, 1), (1, 1)))
    xe = xp[:, 0::2, :]
    xo = xp[:, 1::2, :]

    w1s = jnp.transpose(w1[:, :, 0, :], (2, 0, 1)).reshape(-1, 9)
    h1r = _conv1_call(xe, xo, w1s, b1)

    h1 = jnp.maximum(h1r[..., 0::2], h1r[..., 1::2])
    h1p = jnp.pad(jnp.transpose(h1, (0, 2, 3, 1)),
                  ((0, 0), (1, 1), (1, 1), (0, 0)))

    w2f = w2.reshape(-1, w2.shape[-1])
    feat = _conv2_call(h1p, w2f, b2[None, :])

    return _fc_call(feat.reshape(N, -1), fw1, fb1[None, :], fw2, fb2[None, :])


def kernel(x_nchw, w1, b1, w2, b2, fw1, fb1, fw2, fb2):
    return _forward(x_nchw, w1, b1, w2, b2, fw1, fb1, fw2, fb2)


# speedup vs baseline: 1.0285x; 1.0285x over previous
import functools

import jax
import jax.numpy as jnp
from jax.experimental import pallas as pl
from jax.experimental.pallas import tpu as pltpu


def _conv1_kernel(xe_ref, xo_ref, w1_ref, b1_ref, o_ref, *, H, W, C, B):
    Hh = H // 2
    for n in range(B):
        xe = xe_ref[n]
        xo = xo_ref[n]

        rows_e = [xe[0:Hh], xo[0:Hh], xe[1:Hh + 1]]
        rows_o = [xo[0:Hh], xe[1:Hh + 1], xo[1:Hh + 1]]
        win_e = [[r[:, kw:kw + W] for kw in range(3)] for r in rows_e]
        win_o = [[r[:, kw:kw + W] for kw in range(3)] for r in rows_o]

        for c in range(C):
            acc_e = win_e[0][0] * w1_ref[c, 0]
            acc_o = win_o[0][0] * w1_ref[c, 0]
            for kh in range(3):
                for kw in range(3):
                    if kh == 0 and kw == 0:
                        continue
                    t = kh * 3 + kw
                    acc_e = acc_e + win_e[kh][kw] * w1_ref[c, t]
                    acc_o = acc_o + win_o[kh][kw] * w1_ref[c, t]
            b = b1_ref[c]
            o_ref[n, c] = jnp.maximum(jnp.maximum(acc_e, acc_o) + b, 0.0)


def _conv1_call(xe, xo, w1s, b1, *, B=16):
    N, Hs, Wp = xe.shape
    H, W = (Hs - 1) * 2, Wp - 2
    C = w1s.shape[0]
    B = min(B, N)
    body = functools.partial(_conv1_kernel, H=H, W=W, C=C, B=B)
    return pl.pallas_call(
        body,
        out_shape=jax.ShapeDtypeStruct((N, C, H // 2, W), jnp.float32),
        grid=(N // B,),
        in_specs=[
            pl.BlockSpec((B, Hs, Wp), lambda n: (n, 0, 0)),
            pl.BlockSpec((B, Hs, Wp), lambda n: (n, 0, 0)),
            pl.BlockSpec(memory_space=pltpu.MemorySpace.SMEM),
            pl.BlockSpec(memory_space=pltpu.MemorySpace.SMEM),
        ],
        out_specs=pl.BlockSpec((B, C, H // 2, W), lambda n: (n, 0, 0, 0)),
        compiler_params=pltpu.CompilerParams(
            dimension_semantics=("parallel",),
        ),
    )(xe, xo, w1s, b1)


def _conv2_kernel(xp_ref, w_ref, b_ref, o_ref, h4_ref, p_ref, y_ref,
                  *, H, W, Cin, Cout, B):
    for n in range(B):
        h4_ref[...] = xp_ref[n].reshape(H + 2, W + 2, Cin)
        for kh in range(3):
            for kw in range(3):
                t = kh * 3 + kw
                p_ref[:, t * Cin:(t + 1) * Cin] = (
                    h4_ref[kh:kh + H, kw:kw + W, :].reshape(H * W, Cin))

        y = jnp.dot(p_ref[...], w_ref[...], preferred_element_type=jnp.float32)
        y_ref[...] = jnp.maximum(y + b_ref[...], 0.0)

        hw2 = (H * W) // 2
        yw = jnp.maximum(y_ref[pl.ds(0, hw2, 2), :],
                         y_ref[pl.ds(1, hw2, 2), :])
        o_ref[n] = jnp.max(yw.reshape(H // 2, 2, W // 2, Cout),
                           axis=1).reshape((H // 2) * (W // 2), Cout)


def _conv2_call(xp_flat, w2f, b2, *, Hp, Cin, B=16):
    N, _, WpC = xp_flat.shape
    Wp = WpC // Cin
    H, W = Hp - 2, Wp - 2
    Cout = w2f.shape[-1]
    B = min(B, N)
    body = functools.partial(_conv2_kernel, H=H, W=W, Cin=Cin, Cout=Cout, B=B)
    P = (H // 2) * (W // 2)
    return pl.pallas_call(
        body,
        out_shape=jax.ShapeDtypeStruct((N, P, Cout), jnp.float32),
        grid=(N // B,),
        in_specs=[
            pl.BlockSpec((B, Hp, WpC), lambda n: (n, 0, 0)),
            pl.BlockSpec((9 * Cin, Cout), lambda n: (0, 0)),
            pl.BlockSpec((1, Cout), lambda n: (0, 0)),
        ],
        out_specs=pl.BlockSpec((B, P, Cout), lambda n: (n, 0, 0)),
        scratch_shapes=[
            pltpu.VMEM((Hp, Wp, Cin), jnp.float32),
            pltpu.VMEM((H * W, 9 * Cin), jnp.float32),
            pltpu.VMEM((H * W, Cout), jnp.float32),
        ],
        compiler_params=pltpu.CompilerParams(
            dimension_semantics=("parallel",),
            vmem_limit_bytes=64 * 1024 * 1024,
        ),
    )(xp_flat, w2f, b2)


def _fc_kernel(x_ref, w1_ref, b1_ref, w2_ref, b2_ref, o_ref, acc_ref):
    k = pl.program_id(0)

    @pl.when(k == 0)
    def _():
        acc_ref[...] = jnp.zeros_like(acc_ref)

    acc_ref[...] += jnp.dot(x_ref[...], w1_ref[...],
                            preferred_element_type=jnp.float32)

    @pl.when(k == pl.num_programs(0) - 1)
    def _():
        h = jnp.maximum(acc_ref[...] + b1_ref[...], 0.0)
        o_ref[...] = (jnp.dot(h, w2_ref[...],
                              preferred_element_type=jnp.float32)
                      + b2_ref[...])


def _fc_call(x, fw1, fb1, fw2, fb2, *, tk=4096):
    N, K = x.shape
    tk = min(tk, K)
    Dh = fw1.shape[1]
    Do = fw2.shape[1]
    return pl.pallas_call(
        _fc_kernel,
        out_shape=jax.ShapeDtypeStruct((N, Do), jnp.float32),
        grid=(K // tk,),
        in_specs=[
            pl.BlockSpec((N, tk), lambda k: (0, k)),
            pl.BlockSpec((tk, Dh), lambda k: (k, 0)),
            pl.BlockSpec((1, Dh), lambda k: (0, 0)),
            pl.BlockSpec((Dh, Do), lambda k: (0, 0)),
            pl.BlockSpec((1, Do), lambda k: (0, 0)),
        ],
        out_specs=pl.BlockSpec((N, Do), lambda k: (0, 0)),
        scratch_shapes=[pltpu.VMEM((N, Dh), jnp.float32)],
        compiler_params=pltpu.CompilerParams(
            dimension_semantics=("arbitrary",),
        ),
    )(x, fw1, fb1, fw2, fb2)


@jax.jit
def _forward(x_nchw, w1, b1, w2, b2, fw1, fb1, fw2, fb2):
    N = x_nchw.shape[0]
    xp = jnp.pad(x_nchw[:, 0], ((0, 0), (1, 1), (1, 1)))
    xe = xp[:, 0::2, :]
    xo = xp[:, 1::2, :]

    w1s = jnp.transpose(w1[:, :, 0, :], (2, 0, 1)).reshape(-1, 9)
    h1r = _conv1_call(xe, xo, w1s, b1)

    h1 = jnp.maximum(h1r[..., 0::2], h1r[..., 1::2])
    h1p = jnp.pad(jnp.transpose(h1, (0, 2, 3, 1)),
                  ((0, 0), (1, 1), (1, 1), (0, 0)))
    h1f = h1p.reshape(N, 66, 66 * 16)

    w2f = w2.reshape(-1, w2.shape[-1])
    feat = _conv2_call(h1f, w2f, b2[None, :], Hp=66, Cin=16)

    return _fc_call(feat.reshape(N, -1), fw1, fb1[None, :], fw2, fb2[None, :])


def kernel(x_nchw, w1, b1, w2, b2, fw1, fb1, fw2, fb2):
    return _forward(x_nchw, w1, b1, w2, b2, fw1, fb1, fw2, fb2)


# speedup vs baseline: 2.4036x; 2.3369x over previous
import functools

import jax
import jax.numpy as jnp
from jax.experimental import pallas as pl
from jax.experimental.pallas import tpu as pltpu


def _conv1_kernel(xe_ref, xo_ref, w1_ref, b1_ref, o_ref, *, H, W, C, B):
    Hh = H // 2
    Wh = W // 2
    jj = jax.lax.broadcasted_iota(jnp.int32, (W, Wh), 0)
    rr = jax.lax.broadcasted_iota(jnp.int32, (W, Wh), 1)
    sel_e = (jj == 2 * rr).astype(jnp.float32)
    sel_o = (jj == 2 * rr + 1).astype(jnp.float32)

    for n in range(B):
        o_ref[n, 0:1, :] = jnp.zeros((1, (Wh + 2) * C), jnp.float32)
        o_ref[n, Hh + 1:Hh + 2, :] = jnp.zeros((1, (Wh + 2) * C), jnp.float32)
        o_ref[n, :, 0:C] = jnp.zeros((Hh + 2, C), jnp.float32)
        o_ref[n, :, (Wh + 1) * C:] = jnp.zeros((Hh + 2, C), jnp.float32)
        xe = xe_ref[n]
        xo = xo_ref[n]

        rows_e = [xe[0:Hh], xo[0:Hh], xe[1:Hh + 1]]
        rows_o = [xo[0:Hh], xe[1:Hh + 1], xo[1:Hh + 1]]
        win_e = [[r[:, kw:kw + W] for kw in range(3)] for r in rows_e]
        win_o = [[r[:, kw:kw + W] for kw in range(3)] for r in rows_o]

        pooled = []
        for c in range(C):
            acc_e = win_e[0][0] * w1_ref[c, 0]
            acc_o = win_o[0][0] * w1_ref[c, 0]
            for kh in range(3):
                for kw in range(3):
                    if kh == 0 and kw == 0:
                        continue
                    t = kh * 3 + kw
                    acc_e = acc_e + win_e[kh][kw] * w1_ref[c, t]
                    acc_o = acc_o + win_o[kh][kw] * w1_ref[c, t]
            b = b1_ref[c]
            z = jnp.maximum(jnp.maximum(acc_e, acc_o) + b, 0.0)
            ze = jnp.dot(z, sel_e, preferred_element_type=jnp.float32)
            zo = jnp.dot(z, sel_o, preferred_element_type=jnp.float32)
            pooled.append(jnp.maximum(ze, zo))

        hwc = jnp.transpose(jnp.stack(pooled, axis=1), (0, 2, 1))
        o_ref[n, 1:Hh + 1, C:(Wh + 1) * C] = hwc.reshape(Hh, Wh * C)


def _conv1_call(xe, xo, w1s, b1, *, B=16):
    N, Hs, Wp = xe.shape
    H, W = (Hs - 1) * 2, Wp - 2
    C = w1s.shape[0]
    B = min(B, N)
    body = functools.partial(_conv1_kernel, H=H, W=W, C=C, B=B)
    Ho, Wo = H // 2 + 2, (W // 2 + 2) * C
    return pl.pallas_call(
        body,
        out_shape=jax.ShapeDtypeStruct((N, Ho, Wo), jnp.float32),
        grid=(N // B,),
        in_specs=[
            pl.BlockSpec((B, Hs, Wp), lambda n: (n, 0, 0)),
            pl.BlockSpec((B, Hs, Wp), lambda n: (n, 0, 0)),
            pl.BlockSpec(memory_space=pltpu.MemorySpace.SMEM),
            pl.BlockSpec(memory_space=pltpu.MemorySpace.SMEM),
        ],
        out_specs=pl.BlockSpec((B, Ho, Wo), lambda n: (n, 0, 0)),
        compiler_params=pltpu.CompilerParams(
            dimension_semantics=("parallel",),
        ),
    )(xe, xo, w1s, b1)


def _conv2_kernel(xp_ref, w_ref, b_ref, o_ref, h4_ref, p_ref, y_ref,
                  *, H, W, Cin, Cout, B):
    for n in range(B):
        h4_ref[...] = xp_ref[n].reshape(H + 2, W + 2, Cin)
        for kh in range(3):
            for kw in range(3):
                t = kh * 3 + kw
                p_ref[:, t * Cin:(t + 1) * Cin] = (
                    h4_ref[kh:kh + H, kw:kw + W, :].reshape(H * W, Cin))

        y = jnp.dot(p_ref[...], w_ref[...], preferred_element_type=jnp.float32)
        y_ref[...] = jnp.maximum(y + b_ref[...], 0.0)

        hw2 = (H * W) // 2
        yw = jnp.maximum(y_ref[pl.ds(0, hw2, 2), :],
                         y_ref[pl.ds(1, hw2, 2), :])
        o_ref[n] = jnp.max(yw.reshape(H // 2, 2, W // 2, Cout),
                           axis=1).reshape((H // 2) * (W // 2), Cout)


def _conv2_call(xp_flat, w2f, b2, *, Hp, Cin, B=16):
    N, _, WpC = xp_flat.shape
    Wp = WpC // Cin
    H, W = Hp - 2, Wp - 2
    Cout = w2f.shape[-1]
    B = min(B, N)
    body = functools.partial(_conv2_kernel, H=H, W=W, Cin=Cin, Cout=Cout, B=B)
    P = (H // 2) * (W // 2)
    return pl.pallas_call(
        body,
        out_shape=jax.ShapeDtypeStruct((N, P, Cout), jnp.float32),
        grid=(N // B,),
        in_specs=[
            pl.BlockSpec((B, Hp, WpC), lambda n: (n, 0, 0)),
            pl.BlockSpec((9 * Cin, Cout), lambda n: (0, 0)),
            pl.BlockSpec((1, Cout), lambda n: (0, 0)),
        ],
        out_specs=pl.BlockSpec((B, P, Cout), lambda n: (n, 0, 0)),
        scratch_shapes=[
            pltpu.VMEM((Hp, Wp, Cin), jnp.float32),
            pltpu.VMEM((H * W, 9 * Cin), jnp.float32),
            pltpu.VMEM((H * W, Cout), jnp.float32),
        ],
        compiler_params=pltpu.CompilerParams(
            dimension_semantics=("parallel",),
            vmem_limit_bytes=64 * 1024 * 1024,
        ),
    )(xp_flat, w2f, b2)


def _fc_kernel(x_ref, w1_ref, b1_ref, w2_ref, b2_ref, o_ref, acc_ref):
    k = pl.program_id(0)

    @pl.when(k == 0)
    def _():
        acc_ref[...] = jnp.zeros_like(acc_ref)

    acc_ref[...] += jnp.dot(x_ref[...], w1_ref[...],
                            preferred_element_type=jnp.float32)

    @pl.when(k == pl.num_programs(0) - 1)
    def _():
        h = jnp.maximum(acc_ref[...] + b1_ref[...], 0.0)
        o_ref[...] = (jnp.dot(h, w2_ref[...],
                              preferred_element_type=jnp.float32)
                      + b2_ref[...])


def _fc_call(x, fw1, fb1, fw2, fb2, *, tk=4096):
    N, K = x.shape
    tk = min(tk, K)
    Dh = fw1.shape[1]
    Do = fw2.shape[1]
    return pl.pallas_call(
        _fc_kernel,
        out_shape=jax.ShapeDtypeStruct((N, Do), jnp.float32),
        grid=(K // tk,),
        in_specs=[
            pl.BlockSpec((N, tk), lambda k: (0, k)),
            pl.BlockSpec((tk, Dh), lambda k: (k, 0)),
            pl.BlockSpec((1, Dh), lambda k: (0, 0)),
            pl.BlockSpec((Dh, Do), lambda k: (0, 0)),
            pl.BlockSpec((1, Do), lambda k: (0, 0)),
        ],
        out_specs=pl.BlockSpec((N, Do), lambda k: (0, 0)),
        scratch_shapes=[pltpu.VMEM((N, Dh), jnp.float32)],
        compiler_params=pltpu.CompilerParams(
            dimension_semantics=("arbitrary",),
        ),
    )(x, fw1, fb1, fw2, fb2)


@jax.jit
def _forward(x_nchw, w1, b1, w2, b2, fw1, fb1, fw2, fb2):
    N = x_nchw.shape[0]
    xp = jnp.pad(x_nchw[:, 0], ((0, 0), (1, 1), (1, 1)))
    xe = xp[:, 0::2, :]
    xo = xp[:, 1::2, :]

    w1s = jnp.transpose(w1[:, :, 0, :], (2, 0, 1)).reshape(-1, 9)
    h1f = _conv1_call(xe, xo, w1s, b1)

    w2f = w2.reshape(-1, w2.shape[-1])
    feat = _conv2_call(h1f, w2f, b2[None, :],
                       Hp=h1f.shape[1], Cin=w1.shape[-1])

    return _fc_call(feat.reshape(N, -1), fw1, fb1[None, :], fw2, fb2[None, :])


def kernel(x_nchw, w1, b1, w2, b2, fw1, fb1, fw2, fb2):
    return _forward(x_nchw, w1, b1, w2, b2, fw1, fb1, fw2, fb2)
